# Initial kernel scaffold; baseline (speedup 1.0000x reference)
import jax
import jax.numpy as jnp
from jax.experimental import pallas as pl


def kernel(feature_map):
    raise NotImplementedError("write your pallas kernel here")


# speedup vs baseline: 2.3256x; 2.3256x over previous
import jax
import jax.numpy as jnp
from jax.experimental import pallas as pl
from jax.experimental.pallas import tpu as pltpu

_LANE = 128


def _mm_body(x_ref, o_ref):
    x = x_ref[...]
    br, s, lane = x.shape
    x1 = pltpu.roll(x, _LANE - 1, 2)
    x2 = pltpu.roll(x, _LANE - 2, 2)
    x3 = pltpu.roll(x, _LANE - 3, 2)
    mx = jnp.maximum(jnp.maximum(x, x1), jnp.maximum(x2, x3))
    mn = jnp.minimum(jnp.minimum(x, x1), jnp.minimum(x2, x3))
    idxf = jnp.where(x >= mx, 0.0,
                     jnp.where(x1 >= mx, 1.0,
                               jnp.where(x2 >= mx, 2.0, 3.0)))
    mn2 = pltpu.roll(mn, 2, 2)
    lanes = jax.lax.broadcasted_iota(jnp.int32, x.shape, 2)
    t = jnp.where((lanes & 3) == 0, idxf, mn2)
    pat = (lanes + lanes) & (_LANE - 1)
    g = jnp.take_along_axis(t, pat, axis=2)
    gr = g.reshape(br, s // 2, 2, lane)
    half = jax.lax.broadcasted_iota(jnp.int32, (br, s // 2, lane), 2) < (_LANE // 2)
    o_ref[...] = jnp.where(half, gr[:, :, 0, :], gr[:, :, 1, :])


def kernel(feature_map):
    C, H = feature_map.shape
    S = H // _LANE
    BR = 8
    x3 = feature_map.reshape(C, S, _LANE)
    out = pl.pallas_call(
        _mm_body,
        grid=(C // BR,),
        in_specs=[pl.BlockSpec((BR, S, _LANE), lambda i: (i, 0, 0))],
        out_specs=pl.BlockSpec((BR, S // 2, _LANE), lambda i: (i, 0, 0)),
        out_shape=jax.ShapeDtypeStruct((C, S // 2, _LANE), feature_map.dtype),
        compiler_params=pltpu.CompilerParams(
            dimension_semantics=("parallel",),
        ),
    )(x3)
    return out.reshape(C, H // 2)


# speedup vs baseline: 3.8445x; 1.6531x over previous
import jax
import jax.numpy as jnp
from jax.experimental import pallas as pl
from jax.experimental.pallas import tpu as pltpu


def _mm_body(x_ref, o_ref):
    x = x_ref[...]
    br, bh = x.shape
    x1 = pltpu.roll(x, bh - 1, 1)
    x2 = pltpu.roll(x, bh - 2, 1)
    x3 = pltpu.roll(x, bh - 3, 1)
    mx = jnp.maximum(jnp.maximum(x, x1), jnp.maximum(x2, x3))
    mn = jnp.minimum(jnp.minimum(x, x1), jnp.minimum(x2, x3))
    idxf = jnp.where(x >= mx, 0.0,
                     jnp.where(x1 >= mx, 1.0,
                               jnp.where(x2 >= mx, 2.0, 3.0)))
    mn2 = pltpu.roll(mn, 2, 1)
    lanes = jax.lax.broadcasted_iota(jnp.int32, x.shape, 1)
    t = jnp.where((lanes & 3) == 0, idxf, mn2)
    lane128 = jax.lax.broadcasted_iota(jnp.int32, (br, 128), 1)
    pat = (lane128 + lane128) & 127
    half = lane128 < 64
    chunks = []
    for s in range(bh // 256):
        g0 = jnp.take_along_axis(t[:, 256 * s:256 * s + 128], pat, axis=1)
        g1 = jnp.take_along_axis(t[:, 256 * s + 128:256 * s + 256], pat, axis=1)
        chunks.append(jnp.where(half, g0, g1))
    o_ref[...] = jnp.concatenate(chunks, axis=1)


def kernel(feature_map):
    C, H = feature_map.shape
    BR = 8
    return pl.pallas_call(
        _mm_body,
        grid=(C // BR,),
        in_specs=[pl.BlockSpec((BR, H), lambda i: (i, 0))],
        out_specs=pl.BlockSpec((BR, H // 2), lambda i: (i, 0)),
        out_shape=jax.ShapeDtypeStruct((C, H // 2), feature_map.dtype),
        compiler_params=pltpu.CompilerParams(
            dimension_semantics=("parallel",),
        ),
    )(feature_map)
